# baseline (speedup 1.0000x reference)
import jax
import jax.numpy as jnp
from jax.experimental import pallas as pl
from jax.experimental.pallas import tpu as pltpu

_BF = jnp.bfloat16


def _moe_block_t(hT, hT_bf, wgT_s, w2_s, ct_s):
    e_num = wgT_s.shape[0]
    lgT = jnp.dot(wgT_s[...], hT_bf, preferred_element_type=jnp.float32)
    pT = jnp.exp(lgT)
    pT_bf = pT.astype(_BF)
    rinvT = 1.0 / jnp.sum(pT, axis=0, keepdims=True)
    zuT = jnp.concatenate(
        [pT_bf[e:e + 1, :] * hT_bf for e in range(e_num)], axis=0)
    huT = (jnp.dot(w2_s[...], zuT, preferred_element_type=jnp.float32)
           + jnp.dot(ct_s[...], pT_bf, preferred_element_type=jnp.float32))
    return huT * rinvT


def _fused_kernel(x_ref, wgA_ref, weA_ref, beA_ref, wgB_ref, weB_ref,
                  beB_ref, out_ref,
                  wgA_s, w2A_s, ctA_s, wgB_s, w2B_s, ctB_s):
    @pl.when(pl.program_id(0) == 0)
    def _prologue():
        for wg_ref, we_ref, be_ref, wg_s, w2_s, ct_s in (
                (wgA_ref, weA_ref, beA_ref, wgA_s, w2A_s, ctA_s),
                (wgB_ref, weB_ref, beB_ref, wgB_s, w2B_s, ctB_s)):
            w = we_ref[...]
            e_num, d_num = w.shape[0], w.shape[1]
            w2_s[...] = jnp.transpose(w, (1, 0, 2)).reshape(
                d_num, e_num * d_num).astype(_BF)
            c = -jnp.sum(be_ref[...][:, None, :] * w, axis=-1)
            ct_s[...] = jnp.transpose(c, (1, 0)).astype(_BF)
            wg_s[...] = jnp.transpose(wg_ref[...], (1, 0)).astype(_BF)

    tb = x_ref.shape[0]
    half = tb // 2
    for lo in (0, half):
        xb = x_ref[lo:lo + half, :]
        xT = jnp.transpose(xb, (1, 0))
        hT = _moe_block_t(xT, xT.astype(_BF), wgA_s, w2A_s, ctA_s)
        oT = _moe_block_t(hT, hT.astype(_BF), wgB_s, w2B_s, ctB_s)
        out_ref[lo:lo + half, :] = jnp.transpose(oT + xT, (1, 0))


def kernel(x, wgA, WeA, beA, wgB, WeB, beB):
    t, d = x.shape
    e = wgA.shape[1]
    ed = e * d

    tb = 4096
    grid = (t // tb,)
    full = lambda shape: pl.BlockSpec(shape, lambda i: tuple(0 for _ in shape))
    layer_scratch = [pltpu.VMEM((e, d), _BF), pltpu.VMEM((d, ed), _BF),
                     pltpu.VMEM((d, e), _BF)]
    return pl.pallas_call(
        _fused_kernel,
        grid=grid,
        in_specs=[
            pl.BlockSpec((tb, d), lambda i: (i, 0)),
            full((d, e)), full((e, d, d)), full((e, d)),
            full((d, e)), full((e, d, d)), full((e, d)),
        ],
        out_specs=pl.BlockSpec((tb, d), lambda i: (i, 0)),
        out_shape=jax.ShapeDtypeStruct((t, d), x.dtype),
        scratch_shapes=layer_scratch + layer_scratch,
    )(x, wgA, WeA, beA, wgB, WeB, beB)


# speedup vs baseline: 1.0730x; 1.0730x over previous
import jax
import jax.numpy as jnp
from jax.experimental import pallas as pl
from jax.experimental.pallas import tpu as pltpu

_BF = jnp.bfloat16


def _moe_block_t(hT, hT_bf, wgT_s, w2_s, ct_s):
    e_num = wgT_s.shape[0]
    lgT = jnp.dot(wgT_s[...], hT_bf, preferred_element_type=jnp.float32)
    pT = jnp.exp(lgT)
    pT_bf = pT.astype(_BF)
    rinvT = 1.0 / jnp.sum(pT, axis=0, keepdims=True)
    zuT = jnp.concatenate(
        [pT_bf[e:e + 1, :] * hT_bf for e in range(e_num)], axis=0)
    huT = (jnp.dot(w2_s[...], zuT, preferred_element_type=jnp.float32)
           + jnp.dot(ct_s[...], pT_bf, preferred_element_type=jnp.float32))
    return huT * rinvT


def _fused_kernel(x_ref, wgA_ref, weA_ref, beA_ref, wgB_ref, weB_ref,
                  beB_ref, out_ref,
                  wgA_s, w2A_s, ctA_s, wgB_s, w2B_s, ctB_s):
    @pl.when(pl.program_id(0) == 0)
    def _prologue():
        for wg_ref, we_ref, be_ref, wg_s, w2_s, ct_s in (
                (wgA_ref, weA_ref, beA_ref, wgA_s, w2A_s, ctA_s),
                (wgB_ref, weB_ref, beB_ref, wgB_s, w2B_s, ctB_s)):
            w = we_ref[...]
            e_num, d_num = w.shape[0], w.shape[1]
            w2_s[...] = jnp.transpose(w, (1, 0, 2)).reshape(
                d_num, e_num * d_num).astype(_BF)
            c = -jnp.sum(be_ref[...][:, None, :] * w, axis=-1)
            ct_s[...] = jnp.transpose(c, (1, 0)).astype(_BF)
            wg_s[...] = jnp.transpose(wg_ref[...], (1, 0)).astype(_BF)

    xb = x_ref[...]
    xT_bf = jnp.transpose(xb.astype(_BF), (1, 0))
    hT = _moe_block_t(xT_bf, xT_bf, wgA_s, w2A_s, ctA_s)
    oT = _moe_block_t(hT, hT.astype(_BF), wgB_s, w2B_s, ctB_s)
    out_ref[...] = jnp.transpose(oT, (1, 0)) + xb


def kernel(x, wgA, WeA, beA, wgB, WeB, beB):
    t, d = x.shape
    e = wgA.shape[1]
    ed = e * d

    tb = 4096
    grid = (t // tb,)
    full = lambda shape: pl.BlockSpec(shape, lambda i: tuple(0 for _ in shape))
    layer_scratch = [pltpu.VMEM((e, d), _BF), pltpu.VMEM((d, ed), _BF),
                     pltpu.VMEM((d, e), _BF)]
    return pl.pallas_call(
        _fused_kernel,
        grid=grid,
        in_specs=[
            pl.BlockSpec((tb, d), lambda i: (i, 0)),
            full((d, e)), full((e, d, d)), full((e, d)),
            full((d, e)), full((e, d, d)), full((e, d)),
        ],
        out_specs=pl.BlockSpec((tb, d), lambda i: (i, 0)),
        out_shape=jax.ShapeDtypeStruct((t, d), x.dtype),
        scratch_shapes=layer_scratch + layer_scratch,
    )(x, wgA, WeA, beA, wgB, WeB, beB)
